# Initial kernel scaffold; baseline (speedup 1.0000x reference)
import jax
import jax.numpy as jnp
from jax.experimental import pallas as pl


def kernel(h, edge_index, params):
    raise NotImplementedError("write your pallas kernel here")


# speedup vs baseline: 9.0677x; 9.0677x over previous
import functools

import jax
import jax.numpy as jnp
from jax import lax
from jax.experimental import pallas as pl
from jax.experimental.pallas import tpu as pltpu
from jax.experimental.pallas import tpu_sc as plsc

N = 10000
E = 160000
S = 15
D = 128
DO = 16
N_LAYERS = 2
CNT = float(S * N)
BN_EPS = 1e-5

NC, NS = 2, 16
C = 80
G = 5
EPT = E // NS
NGRP = EPT // (C * G)
ZR = 80
STRIPE = 640
NB = 2000


def _seg_body(x_hbm, src_hbm, dst_hbm, out_hbm, idx_s, idx_d, rows, zbuf,
              accum, sem):
    core = lax.axis_index("c")
    sub = lax.axis_index("s")
    row0 = sub * STRIPE

    def zrow(i, carry):
        for j in range(D // 16):
            zbuf[i, pl.ds(j * 16, 16)] = jnp.zeros((16,), jnp.float32)
        return carry
    lax.fori_loop(0, ZR, zrow, 0)

    for s in range(S):
        @pl.when(core == (s % NC))
        def _process_slice(s=s):
            for k in range(5):
                pltpu.sync_copy(zbuf, accum.at[pl.ds(row0 + k * ZR, ZR)])
            for k in range(5, 8):
                @pl.when(sub < NS - 1)
                def _(k=k):
                    pltpu.sync_copy(zbuf, accum.at[pl.ds(row0 + k * ZR, ZR)])
            plsc.subcore_barrier()

            def group(g, carry):
                pltpu.sync_copy(src_hbm.at[(s * NS + sub) * NGRP + g], idx_s)
                pltpu.sync_copy(dst_hbm.at[sub * NGRP + g], idx_d)
                for ci in range(G):
                    pltpu.async_copy(x_hbm.at[idx_s.at[ci]], rows, sem).wait()
                    pltpu.sync_copy(rows, accum.at[idx_d.at[ci]], add=True)
                return carry
            lax.fori_loop(0, NGRP, group, 0)
            plsc.subcore_barrier()

            pltpu.sync_copy(accum.at[pl.ds(row0, 400)],
                            out_hbm.at[s].at[pl.ds(row0, 400)])
            @pl.when(sub < NS - 1)
            def _():
                pltpu.sync_copy(accum.at[pl.ds(row0 + 400, 240)],
                                out_hbm.at[s].at[pl.ds(row0 + 400, 240)])


_sc_segment_sum = functools.partial(
    pl.kernel,
    out_type=jax.ShapeDtypeStruct((S, N, D), jnp.float32),
    mesh=plsc.VectorSubcoreMesh(core_axis_name="c", subcore_axis_name="s"),
    scratch_types=[
        pltpu.VMEM((G, C), jnp.int32),
        pltpu.VMEM((G, C), jnp.int32),
        pltpu.VMEM((C, D), jnp.float32),
        pltpu.VMEM((ZR, D), jnp.float32),
        pltpu.VMEM_SHARED((N, D), jnp.float32),
        pltpu.SemaphoreType.DMA,
    ],
)(_seg_body)


def _bn_affine(st_ref, g_ref, b_ref):
    mean = st_ref[0:1, :] / CNT
    var = st_ref[1:2, :] / CNT - mean * mean
    a = g_ref[...] * lax.rsqrt(var + BN_EPS)
    return a, b_ref[...] - mean * a


def _accum_stats(st_ref, t, first):
    @pl.when(first)
    def _():
        st_ref[...] = jnp.zeros_like(st_ref)
    st_ref[0:1, :] += jnp.sum(t, axis=0, keepdims=True)
    st_ref[1:2, :] += jnp.sum(t * t, axis=0, keepdims=True)


def _p1(x, agg, w0, b0, eps):
    def body(eps_ref, x_ref, agg_ref, w_ref, b_ref, t_ref, st_ref, pool_ref):
        sidx = pl.program_id(0)
        bidx = pl.program_id(1)
        xb = x_ref[0]
        rst = (1.0 + eps_ref[0]) * xb + agg_ref[0]
        t = jnp.dot(rst, w_ref[...], preferred_element_type=jnp.float32)
        t += b_ref[...]
        t_ref[0] = t
        _accum_stats(st_ref, t, (sidx == 0) & (bidx == 0))
        @pl.when(bidx == 0)
        def _():
            pool_ref[...] = jnp.zeros_like(pool_ref)
        pool_ref[0] += jnp.sum(xb, axis=0, keepdims=True)

    return pl.pallas_call(
        body,
        grid=(S, N // NB),
        in_specs=[
            pl.BlockSpec(memory_space=pltpu.SMEM),
            pl.BlockSpec((1, NB, D), lambda s, b: (s, b, 0)),
            pl.BlockSpec((1, NB, D), lambda s, b: (s, b, 0)),
            pl.BlockSpec((D, D), lambda s, b: (0, 0)),
            pl.BlockSpec((1, D), lambda s, b: (0, 0)),
        ],
        out_specs=[
            pl.BlockSpec((1, NB, D), lambda s, b: (s, b, 0)),
            pl.BlockSpec((2, D), lambda s, b: (0, 0)),
            pl.BlockSpec((1, 1, D), lambda s, b: (s, 0, 0)),
        ],
        out_shape=[
            jax.ShapeDtypeStruct((S, N, D), jnp.float32),
            jax.ShapeDtypeStruct((2, D), jnp.float32),
            jax.ShapeDtypeStruct((S, 1, D), jnp.float32),
        ],
    )(eps.reshape(1), x.reshape(S, N, D), agg.reshape(S, N, D), w0,
      b0.reshape(1, D))


def _p2(t, st_t, g, b, w1, b1):
    def body(t_ref, st_ref, g_ref, b_ref, w_ref, b1_ref, v_ref, stv_ref):
        sidx = pl.program_id(0)
        bidx = pl.program_id(1)
        a, c = _bn_affine(st_ref, g_ref, b_ref)
        u = jnp.maximum(t_ref[0] * a + c, 0.0)
        v = jnp.dot(u, w_ref[...], preferred_element_type=jnp.float32)
        v += b1_ref[...]
        v_ref[0] = v
        _accum_stats(stv_ref, v, (sidx == 0) & (bidx == 0))

    return pl.pallas_call(
        body,
        grid=(S, N // NB),
        in_specs=[
            pl.BlockSpec((1, NB, D), lambda s, b: (s, b, 0)),
            pl.BlockSpec((2, D), lambda s, b: (0, 0)),
            pl.BlockSpec((1, D), lambda s, b: (0, 0)),
            pl.BlockSpec((1, D), lambda s, b: (0, 0)),
            pl.BlockSpec((D, D), lambda s, b: (0, 0)),
            pl.BlockSpec((1, D), lambda s, b: (0, 0)),
        ],
        out_specs=[
            pl.BlockSpec((1, NB, D), lambda s, b: (s, b, 0)),
            pl.BlockSpec((2, D), lambda s, b: (0, 0)),
        ],
        out_shape=[
            jax.ShapeDtypeStruct((S, N, D), jnp.float32),
            jax.ShapeDtypeStruct((2, D), jnp.float32),
        ],
    )(t, st_t, g.reshape(1, D), b.reshape(1, D), w1, b1.reshape(1, D))


def _p3(v, st_v, g, b):
    def body(v_ref, st_ref, g_ref, b_ref, stw_ref):
        sidx = pl.program_id(0)
        bidx = pl.program_id(1)
        a, c = _bn_affine(st_ref, g_ref, b_ref)
        w = jnp.maximum(v_ref[0] * a + c, 0.0)
        _accum_stats(stw_ref, w, (sidx == 0) & (bidx == 0))

    return pl.pallas_call(
        body,
        grid=(S, N // NB),
        in_specs=[
            pl.BlockSpec((1, NB, D), lambda s, b: (s, b, 0)),
            pl.BlockSpec((2, D), lambda s, b: (0, 0)),
            pl.BlockSpec((1, D), lambda s, b: (0, 0)),
            pl.BlockSpec((1, D), lambda s, b: (0, 0)),
        ],
        out_specs=pl.BlockSpec((2, D), lambda s, b: (0, 0)),
        out_shape=jax.ShapeDtypeStruct((2, D), jnp.float32),
    )(v, st_v, g.reshape(1, D), b.reshape(1, D))


def _p4(v, st_v, g1, b1, st_w, g2, b2):
    def body(v_ref, stv_ref, g1_ref, b1_ref, stw_ref, g2_ref, b2_ref,
             hc_ref, pool_ref):
        bidx = pl.program_id(1)
        a1, c1 = _bn_affine(stv_ref, g1_ref, b1_ref)
        w = jnp.maximum(v_ref[0] * a1 + c1, 0.0)
        a2, c2 = _bn_affine(stw_ref, g2_ref, b2_ref)
        hc = jnp.maximum(w * a2 + c2, 0.0)
        hc_ref[0] = hc
        @pl.when(bidx == 0)
        def _():
            pool_ref[...] = jnp.zeros_like(pool_ref)
        pool_ref[0] += jnp.sum(hc, axis=0, keepdims=True)

    return pl.pallas_call(
        body,
        grid=(S, N // NB),
        in_specs=[
            pl.BlockSpec((1, NB, D), lambda s, b: (s, b, 0)),
            pl.BlockSpec((2, D), lambda s, b: (0, 0)),
            pl.BlockSpec((1, D), lambda s, b: (0, 0)),
            pl.BlockSpec((1, D), lambda s, b: (0, 0)),
            pl.BlockSpec((2, D), lambda s, b: (0, 0)),
            pl.BlockSpec((1, D), lambda s, b: (0, 0)),
            pl.BlockSpec((1, D), lambda s, b: (0, 0)),
        ],
        out_specs=[
            pl.BlockSpec((1, NB, D), lambda s, b: (s, b, 0)),
            pl.BlockSpec((1, 1, D), lambda s, b: (s, 0, 0)),
        ],
        out_shape=[
            jax.ShapeDtypeStruct((S, N, D), jnp.float32),
            jax.ShapeDtypeStruct((S, 1, D), jnp.float32),
        ],
    )(v, st_v, g1.reshape(1, D), b1.reshape(1, D), st_w, g2.reshape(1, D),
      b2.reshape(1, D))


def _tail(pools, wps, bsum, fcw, fcb):
    def body(p0_ref, p1_ref, p2_ref, w0_ref, w1_ref, w2_ref, bs_ref, fw_ref,
             fb_ref, out_ref):
        score = jnp.dot(p0_ref[...], w0_ref[...],
                        preferred_element_type=jnp.float32)
        score += jnp.dot(p1_ref[...], w1_ref[...],
                         preferred_element_type=jnp.float32)
        score += jnp.dot(p2_ref[...], w2_ref[...],
                         preferred_element_type=jnp.float32)
        score += bs_ref[...]
        acc = jnp.zeros((S, DO - 9), jnp.float32)
        for q in range(10):
            acc += score[:, q:q + 7] * fw_ref[:, q:q + 1]
        out_ref[...] = jnp.sum(acc, axis=0, keepdims=True) + fb_ref[...]

    return pl.pallas_call(
        body,
        out_shape=jax.ShapeDtypeStruct((1, DO - 9), jnp.float32),
    )(pools[0].reshape(S, D), pools[1].reshape(S, D), pools[2].reshape(S, D),
      wps[0], wps[1], wps[2],
      bsum.reshape(1, DO), fcw.reshape(S, 10), fcb.reshape(1, 1))


def kernel(h, edge_index, params):
    x = jnp.transpose(h[0], (1, 2, 0)).reshape(S * N, D)
    src = edge_index[0]
    dst = edge_index[1]
    offs = (jnp.arange(S, dtype=jnp.int32) * N).reshape(S, 1, 1, 1)
    src2 = (src.reshape(1, NS * NGRP, G, C) + offs).reshape(S * NS * NGRP,
                                                            G, C)
    dst2 = dst.reshape(NS * NGRP, G, C)

    pooled = []
    for l in range(N_LAYERS):
        agg = _sc_segment_sum(x, src2, dst2).reshape(S * N, D)
        t, st_t, pool_x = _p1(x, agg, params['W0_%d' % l],
                              params['b0_%d' % l], params['eps_%d' % l])
        if l == 0:
            pooled.append(pool_x)
        v, st_v = _p2(t, st_t, params['bn_mlp_g_%d' % l],
                      params['bn_mlp_b_%d' % l], params['W1_%d' % l],
                      params['b1_%d' % l])
        st_w = _p3(v, st_v, params['bn_apply_g_%d' % l],
                   params['bn_apply_b_%d' % l])
        xs, pool_hc = _p4(v, st_v, params['bn_apply_g_%d' % l],
                          params['bn_apply_b_%d' % l], st_w,
                          params['bn_out_g_%d' % l], params['bn_out_b_%d' % l])
        x = xs.reshape(S * N, D)
        pooled.append(pool_hc)

    bsum = params['bp_0'] + params['bp_1'] + params['bp_2']
    return _tail(pooled, [params['Wp_0'], params['Wp_1'], params['Wp_2']],
                 bsum, params['fc_w'], params['fc_b'])
